# baseline (speedup 1.0000x reference)
import functools

import jax
import jax.numpy as jnp
from jax import lax
from jax.experimental import pallas as pl
from jax.experimental.pallas import tpu as pltpu
from jax.experimental.pallas import tpu_sc as plsc

B, S, D = 4, 4096, 1024
NW = 32
SCK = 4096
SC_R0 = B * S - SCK
RPW = SCK // NW
R = 32
NCH = RPW // R
JN = D // 16
SC_B = (B * S - 1) // S


def _sc_add_body(x_hbm, ts_hbm, table_hbm, out_hbm,
                 tsv, emb4, buf0, buf1, buf2,
                 gsem, si0, si1, si2, so0, so1, so2):
    cid = lax.axis_index("c")
    sid = lax.axis_index("s")
    wid = cid * 16 + sid
    b = SC_B

    pltpu.sync_copy(ts_hbm, tsv)
    pltpu.async_copy(table_hbm.at[tsv], emb4, gsem).wait()

    row0 = wid * RPW
    bufs = (buf0, buf1, buf2)
    sins = (si0, si1, si2)
    souts = (so0, so1, so2)

    def start_in(c):
        s = c % 3
        return pltpu.async_copy(
            x_hbm.at[pl.ds(SC_R0 + row0 + c * R, R)], bufs[s], sins[s])

    def start_out(c):
        s = c % 3
        return pltpu.async_copy(
            bufs[s], out_hbm.at[pl.ds(row0 + c * R, R)], souts[s])

    def compute(c):
        buf = bufs[c % 3]

        def row(r, carry):
            for j in range(JN):
                sl = pl.ds(j * 16, 16)
                buf[r, sl] = buf[r, sl] + emb4[SC_B, sl]
            return carry

        lax.fori_loop(0, R, row, 0)

    hin = {0: start_in(0), 1: start_in(1)}
    hout = {}
    for c in range(NCH):
        hin[c].wait()
        compute(c)
        hout[c] = start_out(c)
        nxt = c + 2
        if nxt < NCH:
            if nxt - 3 >= 0:
                hout[nxt - 3].wait()
            hin[nxt] = start_in(nxt)
    for c in range(max(0, NCH - 3), NCH):
        hout[c].wait()


def _sc_add(x2, ts, table):
    mesh = plsc.VectorSubcoreMesh(core_axis_name="c", subcore_axis_name="s")
    f = functools.partial(
        pl.kernel, mesh=mesh,
        out_type=jax.ShapeDtypeStruct((SCK, D), jnp.float32),
        scratch_types=[
            pltpu.VMEM((4,), jnp.int32),
            pltpu.VMEM((4, D), jnp.float32),
            pltpu.VMEM((R, D), jnp.float32),
            pltpu.VMEM((R, D), jnp.float32),
            pltpu.VMEM((R, D), jnp.float32),
            pltpu.SemaphoreType.DMA,
            pltpu.SemaphoreType.DMA,
            pltpu.SemaphoreType.DMA,
            pltpu.SemaphoreType.DMA,
            pltpu.SemaphoreType.DMA,
            pltpu.SemaphoreType.DMA,
            pltpu.SemaphoreType.DMA,
        ],
    )(_sc_add_body)
    return f(x2, ts, table)


CH = 512
NCH_TC = SC_R0 // CH
CPB = S // CH
NBUF = 4


def _tc_ring_body(ts_ref, x_hbm, table_hbm, out_hbm, emb, buf, esem, sin, sout):
    for i in range(B):
        pltpu.make_async_copy(table_hbm.at[pl.ds(ts_ref[i], 1)],
                              emb.at[pl.ds(i, 1)], esem).start()
    for i in range(B):
        pltpu.make_async_copy(table_hbm.at[pl.ds(ts_ref[i], 1)],
                              emb.at[pl.ds(i, 1)], esem).wait()

    def in_copy(c, slot):
        return pltpu.make_async_copy(
            x_hbm.at[pl.ds(c * CH, CH)], buf.at[slot], sin.at[slot])

    def out_copy(c, slot):
        return pltpu.make_async_copy(
            buf.at[slot], out_hbm.at[pl.ds(c * CH, CH)], sout.at[slot])

    for c in range(NBUF - 1):
        in_copy(c, c).start()

    def step(c0, carry):
        for k in range(NBUF):
            c = c0 * NBUF + k
            cn = c + NBUF - 1
            kn = (k + NBUF - 1) % NBUF

            @pl.when(cn < NCH_TC)
            def _():
                @pl.when(c >= 1)
                def _():
                    out_copy(c - 1, kn).wait()

                in_copy(cn, kn).start()

            in_copy(c, k).wait()
            b = lax.div(c, CPB)
            buf[k] = buf[k] + emb[pl.ds(b, 1), :]
            out_copy(c, k).start()
        return carry

    lax.fori_loop(0, NCH_TC // NBUF, step, 0)
    for k in range(NBUF):
        cc = NCH_TC - NBUF + k
        out_copy(cc, cc % NBUF).wait()


def _tc_add(x2, ts, embed_table):
    return pl.pallas_call(
        _tc_ring_body,
        in_specs=[
            pl.BlockSpec(memory_space=pltpu.SMEM),
            pl.BlockSpec(memory_space=pl.ANY),
            pl.BlockSpec(memory_space=pl.ANY),
        ],
        out_specs=pl.BlockSpec(memory_space=pl.ANY),
        scratch_shapes=[
            pltpu.VMEM((B, D), jnp.float32),
            pltpu.VMEM((NBUF, CH, D), jnp.float32),
            pltpu.SemaphoreType.DMA,
            pltpu.SemaphoreType.DMA((NBUF,)),
            pltpu.SemaphoreType.DMA((NBUF,)),
        ],
        out_shape=jax.ShapeDtypeStruct((SC_R0, D), jnp.float32),
    )(ts, x2, embed_table)


def kernel(x, timestep, embed_table):
    ts = timestep.astype(jnp.int32)
    x2 = x.reshape(B * S, D)
    out_tc = _tc_add(x2, ts, embed_table)
    out_sc = _sc_add(x2, ts, embed_table)
    return jnp.concatenate([out_tc, out_sc], axis=0).reshape(B, S, D)


# speedup vs baseline: 2.6281x; 2.6281x over previous
import functools

import jax
import jax.numpy as jnp
from jax import lax
from jax.experimental import pallas as pl
from jax.experimental.pallas import tpu as pltpu
from jax.experimental.pallas import tpu_sc as plsc

B, S, D = 4, 4096, 1024
NW = 32
SCK = 4096
SC_R0 = B * S - SCK
RPW = SCK // NW
R = 32
NCH = RPW // R
JN = D // 16
SC_B = (B * S - 1) // S


def _sc_add_body(x_hbm, ts_hbm, table_hbm, out_hbm,
                 tsv, emb4, buf0, buf1, buf2,
                 gsem, si0, si1, si2, so0, so1, so2):
    cid = lax.axis_index("c")
    sid = lax.axis_index("s")
    wid = cid * 16 + sid
    b = SC_B

    pltpu.sync_copy(ts_hbm, tsv)
    pltpu.async_copy(table_hbm.at[tsv], emb4, gsem).wait()

    row0 = wid * RPW
    bufs = (buf0, buf1, buf2)
    sins = (si0, si1, si2)
    souts = (so0, so1, so2)

    def start_in(c):
        s = c % 3
        return pltpu.async_copy(
            x_hbm.at[pl.ds(SC_R0 + row0 + c * R, R)], bufs[s], sins[s])

    def start_out(c):
        s = c % 3
        return pltpu.async_copy(
            bufs[s], out_hbm.at[pl.ds(row0 + c * R, R)], souts[s])

    def compute(c):
        buf = bufs[c % 3]

        def row(r, carry):
            for j in range(JN):
                sl = pl.ds(j * 16, 16)
                buf[r, sl] = buf[r, sl] + emb4[SC_B, sl]
            return carry

        lax.fori_loop(0, R, row, 0)

    hin = {0: start_in(0), 1: start_in(1)}
    hout = {}
    for c in range(NCH):
        hin[c].wait()
        compute(c)
        hout[c] = start_out(c)
        nxt = c + 2
        if nxt < NCH:
            if nxt - 3 >= 0:
                hout[nxt - 3].wait()
            hin[nxt] = start_in(nxt)
    for c in range(max(0, NCH - 3), NCH):
        hout[c].wait()


def _sc_add(x2, ts, table):
    mesh = plsc.VectorSubcoreMesh(core_axis_name="c", subcore_axis_name="s")
    f = functools.partial(
        pl.kernel, mesh=mesh,
        out_type=jax.ShapeDtypeStruct((SCK, D), jnp.float32),
        scratch_types=[
            pltpu.VMEM((4,), jnp.int32),
            pltpu.VMEM((4, D), jnp.float32),
            pltpu.VMEM((R, D), jnp.float32),
            pltpu.VMEM((R, D), jnp.float32),
            pltpu.VMEM((R, D), jnp.float32),
            pltpu.SemaphoreType.DMA,
            pltpu.SemaphoreType.DMA,
            pltpu.SemaphoreType.DMA,
            pltpu.SemaphoreType.DMA,
            pltpu.SemaphoreType.DMA,
            pltpu.SemaphoreType.DMA,
            pltpu.SemaphoreType.DMA,
        ],
    )(_sc_add_body)
    return f(x2, ts, table)


NBUF = 3
MAXCH = 2048
SUB = 256
TC_ROWS = B * S


def _tc_schedule(total_rows):
    head = [256, 256, 256, 256, 1536, 1536]
    tail = [1536, 1536, 256, 256, 256, 256]
    nb = total_rows // S
    chunks = []
    for b in range(nb):
        if b == 0:
            sizes = head
        elif b == nb - 1:
            sizes = tail
        else:
            sizes = [2048, 2048]
        r = b * S
        for n in sizes:
            chunks.append((r, n, b))
            r += n
    return chunks


def _tc_ring_body(ts_ref, x_hbm, table_hbm, out_hbm, emb,
                  buf0, buf1, buf2, esem, sin, sout):
    for i in range(B):
        pltpu.make_async_copy(table_hbm.at[pl.ds(ts_ref[i], 1)],
                              emb.at[pl.ds(i, 1)], esem).start()
    for i in range(B):
        pltpu.make_async_copy(table_hbm.at[pl.ds(ts_ref[i], 1)],
                              emb.at[pl.ds(i, 1)], esem).wait()

    chunks = _tc_schedule(TC_ROWS)
    nch = len(chunks)
    bufs = (buf0, buf1, buf2)

    def in_copy(c):
        r0, n, _ = chunks[c]
        s = c % NBUF
        return pltpu.make_async_copy(
            x_hbm.at[pl.ds(r0, n)], bufs[s].at[pl.ds(0, n)], sin.at[s])

    def out_copy(c):
        r0, n, _ = chunks[c]
        s = c % NBUF
        return pltpu.make_async_copy(
            bufs[s].at[pl.ds(0, n)], out_hbm.at[pl.ds(r0, n)], sout.at[s])

    def compute(c):
        _, n, b = chunks[c]
        buf = bufs[c % NBUF]
        e = emb[pl.ds(b, 1), :]
        if n <= SUB:
            buf[pl.ds(0, n)] = buf[pl.ds(0, n)] + e
        else:
            def sub(i, carry):
                sl = pl.ds(i * SUB, SUB)
                buf[sl] = buf[sl] + e
                return carry
            lax.fori_loop(0, n // SUB, sub, 0)

    hin = {0: in_copy(0), 1: in_copy(1)}
    hin[0].start()
    hin[1].start()
    hout = {}
    for c in range(nch):
        hin[c].wait()
        compute(c)
        hout[c] = out_copy(c)
        hout[c].start()
        nxt = c + NBUF - 1
        if nxt < nch:
            if nxt - NBUF >= 0:
                hout[nxt - NBUF].wait()
            hin[nxt] = in_copy(nxt)
            hin[nxt].start()
    for c in range(max(0, nch - NBUF), nch):
        hout[c].wait()


def _tc_add(x2, ts, embed_table):
    return pl.pallas_call(
        _tc_ring_body,
        in_specs=[
            pl.BlockSpec(memory_space=pltpu.SMEM),
            pl.BlockSpec(memory_space=pl.ANY),
            pl.BlockSpec(memory_space=pl.ANY),
        ],
        out_specs=pl.BlockSpec(memory_space=pl.ANY),
        scratch_shapes=[
            pltpu.VMEM((B, D), jnp.float32),
            pltpu.VMEM((MAXCH, D), jnp.float32),
            pltpu.VMEM((MAXCH, D), jnp.float32),
            pltpu.VMEM((MAXCH, D), jnp.float32),
            pltpu.SemaphoreType.DMA,
            pltpu.SemaphoreType.DMA((NBUF,)),
            pltpu.SemaphoreType.DMA((NBUF,)),
        ],
        out_shape=jax.ShapeDtypeStruct((TC_ROWS, D), jnp.float32),
    )(ts, x2, embed_table)


def kernel(x, timestep, embed_table):
    ts = timestep.astype(jnp.int32)
    x2 = x.reshape(B * S, D)
    return _tc_add(x2, ts, embed_table).reshape(B, S, D)


# speedup vs baseline: 2.7137x; 1.0326x over previous
import functools

import jax
import jax.numpy as jnp
from jax import lax
from jax.experimental import pallas as pl
from jax.experimental.pallas import tpu as pltpu
from jax.experimental.pallas import tpu_sc as plsc

B, S, D = 4, 4096, 1024
NW = 32
SCK = 4096
SC_R0 = B * S - SCK
RPW = SCK // NW
R = 32
NCH = RPW // R
JN = D // 16
SC_B = (B * S - 1) // S


def _sc_add_body(x_hbm, ts_hbm, table_hbm, out_hbm,
                 tsv, emb4, buf0, buf1, buf2,
                 gsem, si0, si1, si2, so0, so1, so2):
    cid = lax.axis_index("c")
    sid = lax.axis_index("s")
    wid = cid * 16 + sid
    b = SC_B

    pltpu.sync_copy(ts_hbm, tsv)
    pltpu.async_copy(table_hbm.at[tsv], emb4, gsem).wait()

    row0 = wid * RPW
    bufs = (buf0, buf1, buf2)
    sins = (si0, si1, si2)
    souts = (so0, so1, so2)

    def start_in(c):
        s = c % 3
        return pltpu.async_copy(
            x_hbm.at[pl.ds(SC_R0 + row0 + c * R, R)], bufs[s], sins[s])

    def start_out(c):
        s = c % 3
        return pltpu.async_copy(
            bufs[s], out_hbm.at[pl.ds(row0 + c * R, R)], souts[s])

    def compute(c):
        buf = bufs[c % 3]

        def row(r, carry):
            for j in range(JN):
                sl = pl.ds(j * 16, 16)
                buf[r, sl] = buf[r, sl] + emb4[SC_B, sl]
            return carry

        lax.fori_loop(0, R, row, 0)

    hin = {0: start_in(0), 1: start_in(1)}
    hout = {}
    for c in range(NCH):
        hin[c].wait()
        compute(c)
        hout[c] = start_out(c)
        nxt = c + 2
        if nxt < NCH:
            if nxt - 3 >= 0:
                hout[nxt - 3].wait()
            hin[nxt] = start_in(nxt)
    for c in range(max(0, NCH - 3), NCH):
        hout[c].wait()


def _sc_add(x2, ts, table):
    mesh = plsc.VectorSubcoreMesh(core_axis_name="c", subcore_axis_name="s")
    f = functools.partial(
        pl.kernel, mesh=mesh,
        out_type=jax.ShapeDtypeStruct((SCK, D), jnp.float32),
        scratch_types=[
            pltpu.VMEM((4,), jnp.int32),
            pltpu.VMEM((4, D), jnp.float32),
            pltpu.VMEM((R, D), jnp.float32),
            pltpu.VMEM((R, D), jnp.float32),
            pltpu.VMEM((R, D), jnp.float32),
            pltpu.SemaphoreType.DMA,
            pltpu.SemaphoreType.DMA,
            pltpu.SemaphoreType.DMA,
            pltpu.SemaphoreType.DMA,
            pltpu.SemaphoreType.DMA,
            pltpu.SemaphoreType.DMA,
            pltpu.SemaphoreType.DMA,
        ],
    )(_sc_add_body)
    return f(x2, ts, table)


NBUF = 4
MAXCH = 2048
SUB = 256
TC_ROWS = B * S


def _tc_schedule(total_rows):
    head = [128, 128, 256, 512, 1024, 2048]
    tail = [2048, 1024, 512, 256, 128, 128]
    nb = total_rows // S
    chunks = []
    for b in range(nb):
        if b == 0:
            sizes = head
        elif b == nb - 1:
            sizes = tail
        else:
            sizes = [2048, 2048]
        r = b * S
        for n in sizes:
            chunks.append((r, n, b))
            r += n
    return chunks


def _tc_ring_body(ts_ref, x_hbm, table_hbm, out_hbm, emb,
                  buf0, buf1, buf2, buf3, esem, sin, sout):
    for i in range(B):
        pltpu.make_async_copy(table_hbm.at[pl.ds(ts_ref[i], 1)],
                              emb.at[pl.ds(i, 1)], esem).start()
    for i in range(B):
        pltpu.make_async_copy(table_hbm.at[pl.ds(ts_ref[i], 1)],
                              emb.at[pl.ds(i, 1)], esem).wait()

    chunks = _tc_schedule(TC_ROWS)
    nch = len(chunks)
    bufs = (buf0, buf1, buf2, buf3)

    def in_copy(c):
        r0, n, _ = chunks[c]
        s = c % NBUF
        return pltpu.make_async_copy(
            x_hbm.at[pl.ds(r0, n)], bufs[s].at[pl.ds(0, n)], sin.at[s])

    def out_copy(c):
        r0, n, _ = chunks[c]
        s = c % NBUF
        return pltpu.make_async_copy(
            bufs[s].at[pl.ds(0, n)], out_hbm.at[pl.ds(r0, n)], sout.at[s])

    def compute(c):
        _, n, b = chunks[c]
        buf = bufs[c % NBUF]
        e = emb[pl.ds(b, 1), :]
        if n <= SUB:
            buf[pl.ds(0, n)] = buf[pl.ds(0, n)] + e
        else:
            def sub(i, carry):
                sl = pl.ds(i * SUB, SUB)
                buf[sl] = buf[sl] + e
                return carry
            lax.fori_loop(0, n // SUB, sub, 0)

    hin = {}
    for c in range(NBUF - 1):
        hin[c] = in_copy(c)
        hin[c].start()
    hout = {}
    for c in range(nch):
        hin[c].wait()
        compute(c)
        hout[c] = out_copy(c)
        hout[c].start()
        nxt = c + NBUF - 1
        if nxt < nch:
            if nxt - NBUF >= 0:
                hout[nxt - NBUF].wait()
            hin[nxt] = in_copy(nxt)
            hin[nxt].start()
    for c in range(max(0, nch - NBUF), nch):
        hout[c].wait()


def _tc_add(x2, ts, embed_table):
    return pl.pallas_call(
        _tc_ring_body,
        in_specs=[
            pl.BlockSpec(memory_space=pltpu.SMEM),
            pl.BlockSpec(memory_space=pl.ANY),
            pl.BlockSpec(memory_space=pl.ANY),
        ],
        out_specs=pl.BlockSpec(memory_space=pl.ANY),
        scratch_shapes=[
            pltpu.VMEM((B, D), jnp.float32),
            pltpu.VMEM((MAXCH, D), jnp.float32),
            pltpu.VMEM((MAXCH, D), jnp.float32),
            pltpu.VMEM((MAXCH, D), jnp.float32),
            pltpu.VMEM((MAXCH, D), jnp.float32),
            pltpu.SemaphoreType.DMA,
            pltpu.SemaphoreType.DMA((NBUF,)),
            pltpu.SemaphoreType.DMA((NBUF,)),
        ],
        out_shape=jax.ShapeDtypeStruct((TC_ROWS, D), jnp.float32),
    )(ts, x2, embed_table)


def kernel(x, timestep, embed_table):
    ts = timestep.astype(jnp.int32)
    x2 = x.reshape(B * S, D)
    return _tc_add(x2, ts, embed_table).reshape(B, S, D)


# speedup vs baseline: 2.7230x; 1.0034x over previous
import functools

import jax
import jax.numpy as jnp
from jax import lax
from jax.experimental import pallas as pl
from jax.experimental.pallas import tpu as pltpu
from jax.experimental.pallas import tpu_sc as plsc

B, S, D = 4, 4096, 1024
NW = 32
SCK = 4096
SC_R0 = B * S - SCK
RPW = SCK // NW
R = 32
NCH = RPW // R
JN = D // 16
SC_B = (B * S - 1) // S


def _sc_add_body(x_hbm, ts_hbm, table_hbm, out_hbm,
                 tsv, emb4, buf0, buf1, buf2,
                 gsem, si0, si1, si2, so0, so1, so2):
    cid = lax.axis_index("c")
    sid = lax.axis_index("s")
    wid = cid * 16 + sid
    b = SC_B

    pltpu.sync_copy(ts_hbm, tsv)
    pltpu.async_copy(table_hbm.at[tsv], emb4, gsem).wait()

    row0 = wid * RPW
    bufs = (buf0, buf1, buf2)
    sins = (si0, si1, si2)
    souts = (so0, so1, so2)

    def start_in(c):
        s = c % 3
        return pltpu.async_copy(
            x_hbm.at[pl.ds(SC_R0 + row0 + c * R, R)], bufs[s], sins[s])

    def start_out(c):
        s = c % 3
        return pltpu.async_copy(
            bufs[s], out_hbm.at[pl.ds(row0 + c * R, R)], souts[s])

    def compute(c):
        buf = bufs[c % 3]

        def row(r, carry):
            for j in range(JN):
                sl = pl.ds(j * 16, 16)
                buf[r, sl] = buf[r, sl] + emb4[SC_B, sl]
            return carry

        lax.fori_loop(0, R, row, 0)

    hin = {0: start_in(0), 1: start_in(1)}
    hout = {}
    for c in range(NCH):
        hin[c].wait()
        compute(c)
        hout[c] = start_out(c)
        nxt = c + 2
        if nxt < NCH:
            if nxt - 3 >= 0:
                hout[nxt - 3].wait()
            hin[nxt] = start_in(nxt)
    for c in range(max(0, NCH - 3), NCH):
        hout[c].wait()


def _sc_add(x2, ts, table):
    mesh = plsc.VectorSubcoreMesh(core_axis_name="c", subcore_axis_name="s")
    f = functools.partial(
        pl.kernel, mesh=mesh,
        out_type=jax.ShapeDtypeStruct((SCK, D), jnp.float32),
        scratch_types=[
            pltpu.VMEM((4,), jnp.int32),
            pltpu.VMEM((4, D), jnp.float32),
            pltpu.VMEM((R, D), jnp.float32),
            pltpu.VMEM((R, D), jnp.float32),
            pltpu.VMEM((R, D), jnp.float32),
            pltpu.SemaphoreType.DMA,
            pltpu.SemaphoreType.DMA,
            pltpu.SemaphoreType.DMA,
            pltpu.SemaphoreType.DMA,
            pltpu.SemaphoreType.DMA,
            pltpu.SemaphoreType.DMA,
            pltpu.SemaphoreType.DMA,
        ],
    )(_sc_add_body)
    return f(x2, ts, table)


NBUF = 4
MAXCH = 2048
SUB = 256
TC_ROWS = B * S


def _tc_schedule(total_rows):
    head = [64, 64, 128, 256, 512, 1024, 2048]
    tail = [2048, 1024, 512, 256, 128, 64, 64]
    nb = total_rows // S
    chunks = []
    for b in range(nb):
        if b == 0:
            sizes = head
        elif b == nb - 1:
            sizes = tail
        else:
            sizes = [2048, 2048]
        r = b * S
        for n in sizes:
            chunks.append((r, n, b))
            r += n
    return chunks


def _tc_ring_body(ts_ref, x_hbm, table_hbm, out_hbm, emb,
                  buf0, buf1, buf2, buf3, esem, sin, sout):
    for i in range(B):
        pltpu.make_async_copy(table_hbm.at[pl.ds(ts_ref[i], 1)],
                              emb.at[pl.ds(i, 1)], esem).start()

    chunks = _tc_schedule(TC_ROWS)
    nch = len(chunks)
    bufs = (buf0, buf1, buf2, buf3)

    def in_copy(c):
        r0, n, _ = chunks[c]
        s = c % NBUF
        return pltpu.make_async_copy(
            x_hbm.at[pl.ds(r0, n)], bufs[s].at[pl.ds(0, n)], sin.at[s])

    def out_copy(c):
        r0, n, _ = chunks[c]
        s = c % NBUF
        return pltpu.make_async_copy(
            bufs[s].at[pl.ds(0, n)], out_hbm.at[pl.ds(r0, n)], sout.at[s])

    def compute(c):
        _, n, b = chunks[c]
        buf = bufs[c % NBUF]
        e = emb[pl.ds(b, 1), :]
        if n <= SUB:
            buf[pl.ds(0, n)] = buf[pl.ds(0, n)] + e
        else:
            def sub(i, carry):
                sl = pl.ds(i * SUB, SUB)
                buf[sl] = buf[sl] + e
                return carry
            lax.fori_loop(0, n // SUB, sub, 0)

    hin = {}
    for c in range(NBUF - 1):
        hin[c] = in_copy(c)
        hin[c].start()
    for i in range(B):
        pltpu.make_async_copy(table_hbm.at[pl.ds(ts_ref[i], 1)],
                              emb.at[pl.ds(i, 1)], esem).wait()
    hout = {}
    for c in range(nch):
        hin[c].wait()
        compute(c)
        hout[c] = out_copy(c)
        hout[c].start()
        nxt = c + NBUF - 1
        if nxt < nch:
            if nxt - NBUF >= 0:
                hout[nxt - NBUF].wait()
            hin[nxt] = in_copy(nxt)
            hin[nxt].start()
    for c in range(max(0, nch - NBUF), nch):
        hout[c].wait()


def _tc_add(x2, ts, embed_table):
    return pl.pallas_call(
        _tc_ring_body,
        in_specs=[
            pl.BlockSpec(memory_space=pltpu.SMEM),
            pl.BlockSpec(memory_space=pl.ANY),
            pl.BlockSpec(memory_space=pl.ANY),
        ],
        out_specs=pl.BlockSpec(memory_space=pl.ANY),
        scratch_shapes=[
            pltpu.VMEM((B, D), jnp.float32),
            pltpu.VMEM((MAXCH, D), jnp.float32),
            pltpu.VMEM((MAXCH, D), jnp.float32),
            pltpu.VMEM((MAXCH, D), jnp.float32),
            pltpu.VMEM((MAXCH, D), jnp.float32),
            pltpu.SemaphoreType.DMA,
            pltpu.SemaphoreType.DMA((NBUF,)),
            pltpu.SemaphoreType.DMA((NBUF,)),
        ],
        out_shape=jax.ShapeDtypeStruct((TC_ROWS, D), jnp.float32),
    )(ts, x2, embed_table)


def kernel(x, timestep, embed_table):
    ts = timestep.astype(jnp.int32)
    x2 = x.reshape(B * S, D)
    return _tc_add(x2, ts, embed_table).reshape(B, S, D)


# speedup vs baseline: 2.7631x; 1.0147x over previous
import functools

import jax
import jax.numpy as jnp
from jax import lax
from jax.experimental import pallas as pl
from jax.experimental.pallas import tpu as pltpu
from jax.experimental.pallas import tpu_sc as plsc

B, S, D = 4, 4096, 1024
NW = 32
SCK = 4096
SC_R0 = B * S - SCK
RPW = SCK // NW
R = 32
NCH = RPW // R
JN = D // 16
SC_B = (B * S - 1) // S


def _sc_add_body(x_hbm, ts_hbm, table_hbm, out_hbm,
                 tsv, emb4, buf0, buf1, buf2,
                 gsem, si0, si1, si2, so0, so1, so2):
    cid = lax.axis_index("c")
    sid = lax.axis_index("s")
    wid = cid * 16 + sid
    b = SC_B

    pltpu.sync_copy(ts_hbm, tsv)
    pltpu.async_copy(table_hbm.at[tsv], emb4, gsem).wait()

    row0 = wid * RPW
    bufs = (buf0, buf1, buf2)
    sins = (si0, si1, si2)
    souts = (so0, so1, so2)

    def start_in(c):
        s = c % 3
        return pltpu.async_copy(
            x_hbm.at[pl.ds(SC_R0 + row0 + c * R, R)], bufs[s], sins[s])

    def start_out(c):
        s = c % 3
        return pltpu.async_copy(
            bufs[s], out_hbm.at[pl.ds(row0 + c * R, R)], souts[s])

    def compute(c):
        buf = bufs[c % 3]

        def row(r, carry):
            for j in range(JN):
                sl = pl.ds(j * 16, 16)
                buf[r, sl] = buf[r, sl] + emb4[SC_B, sl]
            return carry

        lax.fori_loop(0, R, row, 0)

    hin = {0: start_in(0), 1: start_in(1)}
    hout = {}
    for c in range(NCH):
        hin[c].wait()
        compute(c)
        hout[c] = start_out(c)
        nxt = c + 2
        if nxt < NCH:
            if nxt - 3 >= 0:
                hout[nxt - 3].wait()
            hin[nxt] = start_in(nxt)
    for c in range(max(0, NCH - 3), NCH):
        hout[c].wait()


def _sc_add(x2, ts, table):
    mesh = plsc.VectorSubcoreMesh(core_axis_name="c", subcore_axis_name="s")
    f = functools.partial(
        pl.kernel, mesh=mesh,
        out_type=jax.ShapeDtypeStruct((SCK, D), jnp.float32),
        scratch_types=[
            pltpu.VMEM((4,), jnp.int32),
            pltpu.VMEM((4, D), jnp.float32),
            pltpu.VMEM((R, D), jnp.float32),
            pltpu.VMEM((R, D), jnp.float32),
            pltpu.VMEM((R, D), jnp.float32),
            pltpu.SemaphoreType.DMA,
            pltpu.SemaphoreType.DMA,
            pltpu.SemaphoreType.DMA,
            pltpu.SemaphoreType.DMA,
            pltpu.SemaphoreType.DMA,
            pltpu.SemaphoreType.DMA,
            pltpu.SemaphoreType.DMA,
        ],
    )(_sc_add_body)
    return f(x2, ts, table)


NBUF = 5
MAXCH = 2048
SUB = 256
TC_ROWS = B * S


def _tc_schedule(total_rows):
    head = [64, 64, 128, 256, 512, 1024, 2048]
    tail = [2048, 1024, 512, 256, 128, 64, 64]
    nb = total_rows // S
    chunks = []
    for b in range(nb):
        if b == 0:
            sizes = head
        elif b == nb - 1:
            sizes = tail
        else:
            sizes = [2048, 2048]
        r = b * S
        for n in sizes:
            chunks.append((r, n, b))
            r += n
    return chunks


def _tc_ring_body(ts_ref, x_hbm, table_hbm, out_hbm, emb,
                  buf0, buf1, buf2, buf3, buf4, esem, sin, sout):
    for i in range(B):
        pltpu.make_async_copy(table_hbm.at[pl.ds(ts_ref[i], 1)],
                              emb.at[pl.ds(i, 1)], esem).start()

    chunks = _tc_schedule(TC_ROWS)
    nch = len(chunks)
    bufs = (buf0, buf1, buf2, buf3, buf4)

    def in_copy(c):
        r0, n, _ = chunks[c]
        s = c % NBUF
        return pltpu.make_async_copy(
            x_hbm.at[pl.ds(r0, n)], bufs[s].at[pl.ds(0, n)], sin.at[s])

    def out_copy(c):
        r0, n, _ = chunks[c]
        s = c % NBUF
        return pltpu.make_async_copy(
            bufs[s].at[pl.ds(0, n)], out_hbm.at[pl.ds(r0, n)], sout.at[s])

    def compute(c):
        _, n, b = chunks[c]
        buf = bufs[c % NBUF]
        e = emb[pl.ds(b, 1), :]
        if n <= SUB:
            buf[pl.ds(0, n)] = buf[pl.ds(0, n)] + e
        else:
            def sub(i, carry):
                sl = pl.ds(i * SUB, SUB)
                buf[sl] = buf[sl] + e
                return carry
            lax.fori_loop(0, n // SUB, sub, 0)

    hin = {}
    for c in range(NBUF - 1):
        hin[c] = in_copy(c)
        hin[c].start()
    for i in range(B):
        pltpu.make_async_copy(table_hbm.at[pl.ds(ts_ref[i], 1)],
                              emb.at[pl.ds(i, 1)], esem).wait()
    hout = {}
    for c in range(nch):
        hin[c].wait()
        compute(c)
        hout[c] = out_copy(c)
        hout[c].start()
        nxt = c + NBUF - 1
        if nxt < nch:
            if nxt - NBUF >= 0:
                hout[nxt - NBUF].wait()
            hin[nxt] = in_copy(nxt)
            hin[nxt].start()
    for c in range(max(0, nch - NBUF), nch):
        hout[c].wait()


def _tc_add(x2, ts, embed_table):
    return pl.pallas_call(
        _tc_ring_body,
        in_specs=[
            pl.BlockSpec(memory_space=pltpu.SMEM),
            pl.BlockSpec(memory_space=pl.ANY),
            pl.BlockSpec(memory_space=pl.ANY),
        ],
        out_specs=pl.BlockSpec(memory_space=pl.ANY),
        scratch_shapes=[
            pltpu.VMEM((B, D), jnp.float32),
            pltpu.VMEM((MAXCH, D), jnp.float32),
            pltpu.VMEM((MAXCH, D), jnp.float32),
            pltpu.VMEM((MAXCH, D), jnp.float32),
            pltpu.VMEM((MAXCH, D), jnp.float32),
            pltpu.VMEM((MAXCH, D), jnp.float32),
            pltpu.SemaphoreType.DMA,
            pltpu.SemaphoreType.DMA((NBUF,)),
            pltpu.SemaphoreType.DMA((NBUF,)),
        ],
        out_shape=jax.ShapeDtypeStruct((TC_ROWS, D), jnp.float32),
    )(ts, x2, embed_table)


def kernel(x, timestep, embed_table):
    ts = timestep.astype(jnp.int32)
    x2 = x.reshape(B * S, D)
    return _tc_add(x2, ts, embed_table).reshape(B, S, D)


# speedup vs baseline: 2.7952x; 1.0116x over previous
import functools

import jax
import jax.numpy as jnp
from jax import lax
from jax.experimental import pallas as pl
from jax.experimental.pallas import tpu as pltpu
from jax.experimental.pallas import tpu_sc as plsc

B, S, D = 4, 4096, 1024
NW = 32
SCK = 4096
SC_R0 = B * S - SCK
RPW = SCK // NW
R = 32
NCH = RPW // R
JN = D // 16
SC_B = (B * S - 1) // S


def _sc_add_body(x_hbm, ts_hbm, table_hbm, out_hbm,
                 tsv, emb4, buf0, buf1, buf2,
                 gsem, si0, si1, si2, so0, so1, so2):
    cid = lax.axis_index("c")
    sid = lax.axis_index("s")
    wid = cid * 16 + sid
    b = SC_B

    pltpu.sync_copy(ts_hbm, tsv)
    pltpu.async_copy(table_hbm.at[tsv], emb4, gsem).wait()

    row0 = wid * RPW
    bufs = (buf0, buf1, buf2)
    sins = (si0, si1, si2)
    souts = (so0, so1, so2)

    def start_in(c):
        s = c % 3
        return pltpu.async_copy(
            x_hbm.at[pl.ds(SC_R0 + row0 + c * R, R)], bufs[s], sins[s])

    def start_out(c):
        s = c % 3
        return pltpu.async_copy(
            bufs[s], out_hbm.at[pl.ds(row0 + c * R, R)], souts[s])

    def compute(c):
        buf = bufs[c % 3]

        def row(r, carry):
            for j in range(JN):
                sl = pl.ds(j * 16, 16)
                buf[r, sl] = buf[r, sl] + emb4[SC_B, sl]
            return carry

        lax.fori_loop(0, R, row, 0)

    hin = {0: start_in(0), 1: start_in(1)}
    hout = {}
    for c in range(NCH):
        hin[c].wait()
        compute(c)
        hout[c] = start_out(c)
        nxt = c + 2
        if nxt < NCH:
            if nxt - 3 >= 0:
                hout[nxt - 3].wait()
            hin[nxt] = start_in(nxt)
    for c in range(max(0, NCH - 3), NCH):
        hout[c].wait()


def _sc_add(x2, ts, table):
    mesh = plsc.VectorSubcoreMesh(core_axis_name="c", subcore_axis_name="s")
    f = functools.partial(
        pl.kernel, mesh=mesh,
        out_type=jax.ShapeDtypeStruct((SCK, D), jnp.float32),
        scratch_types=[
            pltpu.VMEM((4,), jnp.int32),
            pltpu.VMEM((4, D), jnp.float32),
            pltpu.VMEM((R, D), jnp.float32),
            pltpu.VMEM((R, D), jnp.float32),
            pltpu.VMEM((R, D), jnp.float32),
            pltpu.SemaphoreType.DMA,
            pltpu.SemaphoreType.DMA,
            pltpu.SemaphoreType.DMA,
            pltpu.SemaphoreType.DMA,
            pltpu.SemaphoreType.DMA,
            pltpu.SemaphoreType.DMA,
            pltpu.SemaphoreType.DMA,
        ],
    )(_sc_add_body)
    return f(x2, ts, table)


NBUF = 6
MAXCH = 2048
SUB = 256
TC_ROWS = B * S


def _tc_schedule(total_rows):
    head = [32, 32, 64, 128, 256, 512, 1024, 2048]
    tail = [2048, 1024, 512, 256, 128, 64, 32, 32]
    nb = total_rows // S
    chunks = []
    for b in range(nb):
        if b == 0:
            sizes = head
        elif b == nb - 1:
            sizes = tail
        else:
            sizes = [2048, 2048]
        r = b * S
        for n in sizes:
            chunks.append((r, n, b))
            r += n
    return chunks


def _tc_ring_body(ts_ref, x_hbm, table_hbm, out_hbm, emb,
                  buf0, buf1, buf2, buf3, buf4, buf5, esem, sin, sout):
    for i in range(B):
        pltpu.make_async_copy(table_hbm.at[pl.ds(ts_ref[i], 1)],
                              emb.at[pl.ds(i, 1)], esem).start()

    chunks = _tc_schedule(TC_ROWS)
    nch = len(chunks)
    bufs = (buf0, buf1, buf2, buf3, buf4, buf5)

    def in_copy(c):
        r0, n, _ = chunks[c]
        s = c % NBUF
        return pltpu.make_async_copy(
            x_hbm.at[pl.ds(r0, n)], bufs[s].at[pl.ds(0, n)], sin.at[s])

    def out_copy(c):
        r0, n, _ = chunks[c]
        s = c % NBUF
        return pltpu.make_async_copy(
            bufs[s].at[pl.ds(0, n)], out_hbm.at[pl.ds(r0, n)], sout.at[s])

    def compute(c):
        _, n, b = chunks[c]
        buf = bufs[c % NBUF]
        e = emb[pl.ds(b, 1), :]
        if n <= SUB:
            buf[pl.ds(0, n)] = buf[pl.ds(0, n)] + e
        else:
            def sub(i, carry):
                sl = pl.ds(i * SUB, SUB)
                buf[sl] = buf[sl] + e
                return carry
            lax.fori_loop(0, n // SUB, sub, 0)

    hin = {}
    for c in range(NBUF - 1):
        hin[c] = in_copy(c)
        hin[c].start()
    for i in range(B):
        pltpu.make_async_copy(table_hbm.at[pl.ds(ts_ref[i], 1)],
                              emb.at[pl.ds(i, 1)], esem).wait()
    hout = {}
    for c in range(nch):
        hin[c].wait()
        compute(c)
        hout[c] = out_copy(c)
        hout[c].start()
        nxt = c + NBUF - 1
        if nxt < nch:
            if nxt - NBUF >= 0:
                hout[nxt - NBUF].wait()
            hin[nxt] = in_copy(nxt)
            hin[nxt].start()
    for c in range(max(0, nch - NBUF), nch):
        hout[c].wait()


def _tc_add(x2, ts, embed_table):
    return pl.pallas_call(
        _tc_ring_body,
        in_specs=[
            pl.BlockSpec(memory_space=pltpu.SMEM),
            pl.BlockSpec(memory_space=pl.ANY),
            pl.BlockSpec(memory_space=pl.ANY),
        ],
        out_specs=pl.BlockSpec(memory_space=pl.ANY),
        scratch_shapes=[
            pltpu.VMEM((B, D), jnp.float32),
            pltpu.VMEM((MAXCH, D), jnp.float32),
            pltpu.VMEM((MAXCH, D), jnp.float32),
            pltpu.VMEM((MAXCH, D), jnp.float32),
            pltpu.VMEM((MAXCH, D), jnp.float32),
            pltpu.VMEM((MAXCH, D), jnp.float32),
            pltpu.VMEM((MAXCH, D), jnp.float32),
            pltpu.SemaphoreType.DMA,
            pltpu.SemaphoreType.DMA((NBUF,)),
            pltpu.SemaphoreType.DMA((NBUF,)),
        ],
        out_shape=jax.ShapeDtypeStruct((TC_ROWS, D), jnp.float32),
    )(ts, x2, embed_table)


def kernel(x, timestep, embed_table):
    ts = timestep.astype(jnp.int32)
    x2 = x.reshape(B * S, D)
    return _tc_add(x2, ts, embed_table).reshape(B, S, D)
